# baseline (speedup 1.0000x reference)
import functools

import jax
import jax.numpy as jnp
from jax import lax
from jax.experimental import pallas as pl


def _ln(x, g, b):
    m = jnp.mean(x, axis=-1, keepdims=True)
    d = x - m
    v = jnp.mean(d * d, axis=-1, keepdims=True)
    return d * lax.rsqrt(v + 1e-5) * g + b


def _mm(a, b):
    return lax.dot_general(
        a, b, (((a.ndim - 1,), (0,)), ((), ())),
        preferred_element_type=jnp.float32)


def _mmb(a, b):
    return lax.dot_general(
        a.astype(jnp.bfloat16), b,
        (((a.ndim - 1,), (0,)), ((), ())),
        preferred_element_type=jnp.float32)


def _embed_kernel(xr0_ref, w0_ref, b0_ref, xr1_ref, w1_ref, b1_ref,
                  p0_ref, r0_ref, p1_ref, r1_ref):
    p0 = _mm(xr0_ref[:], w0_ref[:]) + b0_ref[:]
    p0_ref[:] = p0
    r0_ref[0] = jnp.sum(jnp.abs(p0), axis=2)
    p1 = _mm(xr1_ref[:], w1_ref[:]) + b1_ref[:]
    p1_ref[:] = p1
    r1_ref[0] = jnp.sum(jnp.abs(p1), axis=2)


def _topk_mask(s, na, w, P):
    anc = jnp.zeros(s.shape, jnp.bool_)
    for _ in range(na):
        m = jnp.max(s)
        sel = s == m
        anc = jnp.logical_or(anc, sel)
        s = jnp.where(sel, -jnp.inf, s)
    ti = lax.broadcasted_iota(jnp.int32, (P, P), 0)
    si = lax.broadcasted_iota(jnp.int32, (P, P), 1)
    allowed = (si <= ti) & (((ti - si) < w) | anc)
    return jnp.where(allowed, 0.0, -1e9).astype(jnp.float32)


def _route_kernel(nv1_ref, nv2_ref, r30_ref, r31_ref,
                  m0_ref, m1_ref, *, cfg0, cfg1):
    a = jnp.maximum(
        jnp.dot(nv1_ref[:].astype(jnp.bfloat16),
                nv2_ref[:].astype(jnp.bfloat16),
                preferred_element_type=jnp.float32),
        0.0)
    mx = jnp.max(a, axis=1, keepdims=True)
    e = jnp.exp(a - mx)
    adp = e / jnp.sum(e, axis=1, keepdims=True)
    imp = jnp.sum(adp, axis=0, keepdims=True)
    impb = imp.astype(jnp.bfloat16)
    for r3_ref, m_ref, cfg in ((r30_ref, m0_ref, cfg0), (r31_ref, m1_ref, cfg1)):
        na, w, P = cfg
        M = r3_ref[0] + r3_ref[1]
        s = jnp.dot(impb, M.astype(jnp.bfloat16),
                    preferred_element_type=jnp.float32)
        m_ref[:] = _topk_mask(s, na, w, P)


def _encode(x, mask, refs, depth):
    (lng1, lnb1, wq, bq, wk, bk, wv, bv, wo, bo,
     lng2, lnb2, w1, b1, w2, b2) = refs
    H = wq.shape[1]
    for d in range(depth):
        h1 = _ln(x, lng1[d], lnb1[d])
        acc = jnp.zeros_like(x)
        for hh in range(H):
            q = _mmb(h1, wq[d, hh]) + bq[d, hh]
            k = _mmb(h1, wk[d, hh]) + bk[d, hh]
            v = _mmb(h1, wv[d, hh]) + bv[d, hh]
            s = lax.dot_general(
                q.astype(jnp.bfloat16), k.astype(jnp.bfloat16),
                (((2,), (2,)), ((0,), (0,))),
                preferred_element_type=jnp.float32) + mask
            m = jnp.max(s, axis=2, keepdims=True)
            e = jnp.exp(s - m)
            r = 1.0 / jnp.sum(e, axis=2, keepdims=True)
            o = lax.dot_general(
                e.astype(jnp.bfloat16), v.astype(jnp.bfloat16),
                (((2,), (1,)), ((0,), (0,))),
                preferred_element_type=jnp.float32) * r
            acc = acc + _mmb(o, wo[d, hh])
        x = x + acc + bo[d]
        h2 = _ln(x, lng2[d], lnb2[d])
        mid = jax.nn.gelu(_mmb(h2, w1[d]) + b1[d])
        x = x + _mmb(mid, w2[d]) + b2[d]
    return x


def _enc_kernel(*refs, depth):
    x_ref, m_ref = refs[:2]
    wrefs = refs[2:18]
    out_ref = refs[18]
    x = _encode(x_ref[:], m_ref[:], wrefs, depth)
    out_ref[0] = jnp.mean(x, axis=1)


def _fuse_kernel(p0_ref, p1_ref, fw_ref, fW_ref, fb_ref, fg_ref, fbt_ref,
                 out_ref):
    w = fw_ref[:]
    e = jnp.exp(w - jnp.max(w))
    sm = e / jnp.sum(e)
    f = p0_ref[:] * sm[0, 0] + p1_ref[:] * sm[0, 1]
    f = _mmb(f, fW_ref[:]) + fb_ref[:]
    out_ref[:] = _ln(f, fg_ref[:], fbt_ref[:])


def _stack_weights(blocks, D, H, dh):
    bf16 = jnp.bfloat16
    scale = 1.0 / (dh ** 0.5)
    lng1 = jnp.stack([b['ln1g'].reshape(1, D) for b in blocks])
    lnb1 = jnp.stack([b['ln1b'].reshape(1, D) for b in blocks])
    wq = jnp.stack([(b['Wq'] * scale).reshape(D, H, dh).transpose(1, 0, 2)
                    for b in blocks]).astype(bf16)
    bq = jnp.stack([(b['bq'] * scale).reshape(H, 1, dh) for b in blocks])
    wk = jnp.stack([b['Wk'].reshape(D, H, dh).transpose(1, 0, 2)
                    for b in blocks]).astype(bf16)
    bk = jnp.stack([b['bk'].reshape(H, 1, dh) for b in blocks])
    wv = jnp.stack([b['Wv'].reshape(D, H, dh).transpose(1, 0, 2)
                    for b in blocks]).astype(bf16)
    bv = jnp.stack([b['bv'].reshape(H, 1, dh) for b in blocks])
    wo = jnp.stack([b['Wo'].reshape(H, dh, D) for b in blocks]).astype(bf16)
    bo = jnp.stack([b['bo'].reshape(1, D) for b in blocks])
    lng2 = jnp.stack([b['ln2g'].reshape(1, D) for b in blocks])
    lnb2 = jnp.stack([b['ln2b'].reshape(1, D) for b in blocks])
    w1 = jnp.stack([b['W1'] for b in blocks]).astype(bf16)
    b1 = jnp.stack([b['b1'].reshape(1, -1) for b in blocks])
    w2 = jnp.stack([b['W2'] for b in blocks]).astype(bf16)
    b2 = jnp.stack([b['b2'].reshape(1, D) for b in blocks])
    return [lng1, lnb1, wq, bq, wk, bk, wv, bv, wo, bo,
            lng2, lnb2, w1, b1, w2, b2]


def _cst_specs(arrs):
    return [pl.BlockSpec(a.shape, lambda t, _n=a.ndim: (0,) * _n)
            for a in arrs]


def _pick_tile(n, cap=32):
    for g in range(cap, 0, -1):
        if n % g == 0:
            return g
    return 1


def kernel(long_history_data, params, epoch):
    del epoch
    x = long_history_data
    Bn, L, N, C = x.shape
    D = params['fuse_W'].shape[0]
    H = 4
    dh = D // H
    patch_sizes = [int(W.shape[0]) // C for W in params['pe_W']]
    BN = Bn * N
    G = _pick_tile(BN)
    T = BN // G

    f32 = jnp.float32
    bf16 = jnp.bfloat16

    xt = jnp.transpose(x, (0, 2, 3, 1))
    xrs, Ps, Ks = [], [], []
    for i, ps in enumerate(patch_sizes):
        P = L // ps
        K = ps * C
        Ps.append(P)
        Ks.append(K)
        xrs.append((xt.reshape(Bn, N, C, P, ps)
                      .transpose(0, 1, 3, 4, 2)
                      .reshape(BN, P, K)).astype(bf16))
    P0, P1 = Ps

    blk = lambda *dims: pl.BlockSpec(dims, lambda t: (t,) + (0,) * (len(dims) - 1))
    cst = lambda *dims: pl.BlockSpec(dims, lambda t: (0,) * len(dims))

    patches0, r0, patches1, r1 = pl.pallas_call(
        _embed_kernel,
        grid=(T,),
        in_specs=[
            blk(G, P0, Ks[0]), cst(Ks[0], D), cst(1, D),
            blk(G, P1, Ks[1]), cst(Ks[1], D), cst(1, D),
        ],
        out_specs=[
            blk(G, P0, D), blk(1, G, P0),
            blk(G, P1, D), blk(1, G, P1),
        ],
        out_shape=[
            jax.ShapeDtypeStruct((BN, P0, D), f32),
            jax.ShapeDtypeStruct((T, G, P0), f32),
            jax.ShapeDtypeStruct((BN, P1, D), f32),
            jax.ShapeDtypeStruct((T, G, P1), f32),
        ],
    )(xrs[0], params['pe_W'][0].astype(bf16),
      params['pe_b'][0].reshape(1, D),
      xrs[1], params['pe_W'][1].astype(bf16),
      params['pe_b'][1].reshape(1, D))

    cfg0 = (max(1, int(0.1 * P0)), max(1, patch_sizes[0] // 4), P0)
    cfg1 = (max(1, int(0.1 * P1)), max(1, patch_sizes[1] // 4), P1)
    mask0, mask1 = pl.pallas_call(
        functools.partial(_route_kernel, cfg0=cfg0, cfg1=cfg1),
        out_shape=[jax.ShapeDtypeStruct((P0, P0), f32),
                   jax.ShapeDtypeStruct((P1, P1), f32)],
    )(params['nodevec1'], params['nodevec2'],
      r0.reshape(Bn, N, P0), r1.reshape(Bn, N, P1))

    wref0 = _stack_weights(params['encoders'][0], D, H, dh)
    wref1 = _stack_weights(params['encoders'][1], D, H, dh)
    fuse_args = [params['fusion_w'].reshape(1, -1),
                 params['fuse_W'].astype(bf16),
                 params['fuse_b'].reshape(1, D),
                 params['fin_g'].reshape(1, D),
                 params['fin_b'].reshape(1, D)]

    depth = len(params['encoders'][0])
    GE = _pick_tile(BN, 18)
    TE = BN // GE
    pooled = []
    for patches, mask, wrefs, P in ((patches0, mask0, wref0, P0),
                                    (patches1, mask1, wref1, P1)):
        pooled.append(pl.pallas_call(
            functools.partial(_enc_kernel, depth=depth),
            grid=(TE,),
            in_specs=[blk(GE, P, D), cst(P, P)] + _cst_specs(wrefs),
            out_specs=blk(1, GE, D),
            out_shape=jax.ShapeDtypeStruct((TE, GE, D), f32),
        )(patches, mask, *wrefs))

    final = pl.pallas_call(
        _fuse_kernel,
        out_shape=jax.ShapeDtypeStruct((TE, GE, D), f32),
    )(pooled[0], pooled[1], *fuse_args)

    return final.reshape(Bn, N, D)


# speedup vs baseline: 1.2993x; 1.2993x over previous
import functools

import jax
import jax.numpy as jnp
from jax import lax
from jax.experimental import pallas as pl


def _ln(x, g, b):
    m = jnp.mean(x, axis=-1, keepdims=True)
    d = x - m
    v = jnp.mean(d * d, axis=-1, keepdims=True)
    return d * lax.rsqrt(v + 1e-5) * g + b


def _mm(a, b):
    return lax.dot_general(
        a, b, (((a.ndim - 1,), (0,)), ((), ())),
        preferred_element_type=jnp.float32)


def _mmb(a, b):
    return lax.dot_general(
        a.astype(jnp.bfloat16), b,
        (((a.ndim - 1,), (0,)), ((), ())),
        preferred_element_type=jnp.float32)


def _embed_kernel(xr0_ref, w0_ref, b0_ref, xr1_ref, w1_ref, b1_ref,
                  p0_ref, r0_ref, p1_ref, r1_ref):
    p0 = _mm(xr0_ref[:], w0_ref[:]) + b0_ref[:]
    p0_ref[:] = p0
    r0_ref[0] = jnp.sum(jnp.abs(p0), axis=2)
    p1 = _mm(xr1_ref[:], w1_ref[:]) + b1_ref[:]
    p1_ref[:] = p1
    r1_ref[0] = jnp.sum(jnp.abs(p1), axis=2)


def _topk_mask(s, na, w, P):
    anc = jnp.zeros(s.shape, jnp.bool_)
    for _ in range(na):
        m = jnp.max(s)
        sel = s == m
        anc = jnp.logical_or(anc, sel)
        s = jnp.where(sel, -jnp.inf, s)
    ti = lax.broadcasted_iota(jnp.int32, (P, P), 0)
    si = lax.broadcasted_iota(jnp.int32, (P, P), 1)
    allowed = (si <= ti) & (((ti - si) < w) | anc)
    return jnp.where(allowed, 0.0, -1e9).astype(jnp.float32)


def _route_kernel(nv1_ref, nv2_ref, r30_ref, r31_ref,
                  m0_ref, m1_ref, *, cfg0, cfg1):
    a = jnp.maximum(
        jnp.dot(nv1_ref[:].astype(jnp.bfloat16),
                nv2_ref[:].astype(jnp.bfloat16),
                preferred_element_type=jnp.float32),
        0.0)
    mx = jnp.max(a, axis=1, keepdims=True)
    e = jnp.exp(a - mx)
    adp = e / jnp.sum(e, axis=1, keepdims=True)
    imp = jnp.sum(adp, axis=0, keepdims=True)
    impb = imp.astype(jnp.bfloat16)
    for r3_ref, m_ref, cfg in ((r30_ref, m0_ref, cfg0), (r31_ref, m1_ref, cfg1)):
        na, w, P = cfg
        M = r3_ref[0] + r3_ref[1]
        s = jnp.dot(impb, M.astype(jnp.bfloat16),
                    preferred_element_type=jnp.float32)
        m_ref[:] = _topk_mask(s, na, w, P)


def _encode(x, mask, refs, depth):
    (lng1, lnb1, wq, bq, wk, bk, wv, bv, wo, bo,
     lng2, lnb2, w1, b1, w2, b2) = refs
    H = wq.shape[1]
    for d in range(depth):
        h1 = _ln(x, lng1[d], lnb1[d])
        acc = jnp.zeros_like(x)
        for hh in range(H):
            q = _mmb(h1, wq[d, hh]) + bq[d, hh]
            k = _mmb(h1, wk[d, hh]) + bk[d, hh]
            v = _mmb(h1, wv[d, hh]) + bv[d, hh]
            s = lax.dot_general(
                q.astype(jnp.bfloat16), k.astype(jnp.bfloat16),
                (((2,), (2,)), ((0,), (0,))),
                preferred_element_type=jnp.float32) + mask
            m = jnp.max(s, axis=2, keepdims=True)
            e = jnp.exp(s - m)
            r = 1.0 / jnp.sum(e, axis=2, keepdims=True)
            o = lax.dot_general(
                e.astype(jnp.bfloat16), v.astype(jnp.bfloat16),
                (((2,), (1,)), ((0,), (0,))),
                preferred_element_type=jnp.float32) * r
            acc = acc + _mmb(o, wo[d, hh])
        x = x + acc + bo[d]
        h2 = _ln(x, lng2[d], lnb2[d])
        mid = jax.nn.gelu(_mmb(h2, w1[d]) + b1[d])
        x = x + _mmb(mid, w2[d]) + b2[d]
    return x


def _enc_kernel(*refs, depth):
    x_ref, m_ref = refs[:2]
    wrefs = refs[2:18]
    out_ref = refs[18]
    x = _encode(x_ref[:], m_ref[:], wrefs, depth)
    out_ref[0] = jnp.mean(x, axis=1)


def _encfuse_kernel(*refs, depth):
    x_ref, m_ref, p0_ref = refs[:3]
    wrefs = refs[3:19]
    fw_ref, fW_ref, fb_ref, fg_ref, fbt_ref = refs[19:24]
    out_ref = refs[24]
    x = _encode(x_ref[:], m_ref[:], wrefs, depth)
    p1 = jnp.mean(x, axis=1)
    w = fw_ref[:]
    e = jnp.exp(w - jnp.max(w))
    sm = e / jnp.sum(e)
    f = p0_ref[0] * sm[0, 0] + p1 * sm[0, 1]
    f = _mmb(f, fW_ref[:]) + fb_ref[:]
    out_ref[0] = _ln(f, fg_ref[:], fbt_ref[:])


def _stack_weights(blocks, D, H, dh):
    bf16 = jnp.bfloat16
    scale = 1.0 / (dh ** 0.5)
    lng1 = jnp.stack([b['ln1g'].reshape(1, D) for b in blocks])
    lnb1 = jnp.stack([b['ln1b'].reshape(1, D) for b in blocks])
    wq = jnp.stack([(b['Wq'] * scale).reshape(D, H, dh).transpose(1, 0, 2)
                    for b in blocks]).astype(bf16)
    bq = jnp.stack([(b['bq'] * scale).reshape(H, 1, dh) for b in blocks])
    wk = jnp.stack([b['Wk'].reshape(D, H, dh).transpose(1, 0, 2)
                    for b in blocks]).astype(bf16)
    bk = jnp.stack([b['bk'].reshape(H, 1, dh) for b in blocks])
    wv = jnp.stack([b['Wv'].reshape(D, H, dh).transpose(1, 0, 2)
                    for b in blocks]).astype(bf16)
    bv = jnp.stack([b['bv'].reshape(H, 1, dh) for b in blocks])
    wo = jnp.stack([b['Wo'].reshape(H, dh, D) for b in blocks]).astype(bf16)
    bo = jnp.stack([b['bo'].reshape(1, D) for b in blocks])
    lng2 = jnp.stack([b['ln2g'].reshape(1, D) for b in blocks])
    lnb2 = jnp.stack([b['ln2b'].reshape(1, D) for b in blocks])
    w1 = jnp.stack([b['W1'] for b in blocks]).astype(bf16)
    b1 = jnp.stack([b['b1'].reshape(1, -1) for b in blocks])
    w2 = jnp.stack([b['W2'] for b in blocks]).astype(bf16)
    b2 = jnp.stack([b['b2'].reshape(1, D) for b in blocks])
    return [lng1, lnb1, wq, bq, wk, bk, wv, bv, wo, bo,
            lng2, lnb2, w1, b1, w2, b2]


def _cst_specs(arrs):
    return [pl.BlockSpec(a.shape, lambda t, _n=a.ndim: (0,) * _n)
            for a in arrs]


def _pick_tile(n, cap=32):
    for g in range(cap, 0, -1):
        if n % g == 0:
            return g
    return 1


def kernel(long_history_data, params, epoch):
    del epoch
    x = long_history_data
    Bn, L, N, C = x.shape
    D = params['fuse_W'].shape[0]
    H = 4
    dh = D // H
    patch_sizes = [int(W.shape[0]) // C for W in params['pe_W']]
    BN = Bn * N
    G = _pick_tile(BN)
    T = BN // G

    f32 = jnp.float32
    bf16 = jnp.bfloat16

    xt = jnp.transpose(x, (0, 2, 3, 1))
    xrs, Ps, Ks = [], [], []
    for i, ps in enumerate(patch_sizes):
        P = L // ps
        K = ps * C
        Ps.append(P)
        Ks.append(K)
        xrs.append((xt.reshape(Bn, N, C, P, ps)
                      .transpose(0, 1, 3, 4, 2)
                      .reshape(BN, P, K)).astype(bf16))
    P0, P1 = Ps

    blk = lambda *dims: pl.BlockSpec(dims, lambda t: (t,) + (0,) * (len(dims) - 1))
    cst = lambda *dims: pl.BlockSpec(dims, lambda t: (0,) * len(dims))

    patches0, r0, patches1, r1 = pl.pallas_call(
        _embed_kernel,
        grid=(T,),
        in_specs=[
            blk(G, P0, Ks[0]), cst(Ks[0], D), cst(1, D),
            blk(G, P1, Ks[1]), cst(Ks[1], D), cst(1, D),
        ],
        out_specs=[
            blk(G, P0, D), blk(1, G, P0),
            blk(G, P1, D), blk(1, G, P1),
        ],
        out_shape=[
            jax.ShapeDtypeStruct((BN, P0, D), f32),
            jax.ShapeDtypeStruct((T, G, P0), f32),
            jax.ShapeDtypeStruct((BN, P1, D), f32),
            jax.ShapeDtypeStruct((T, G, P1), f32),
        ],
    )(xrs[0], params['pe_W'][0].astype(bf16),
      params['pe_b'][0].reshape(1, D),
      xrs[1], params['pe_W'][1].astype(bf16),
      params['pe_b'][1].reshape(1, D))

    cfg0 = (max(1, int(0.1 * P0)), max(1, patch_sizes[0] // 4), P0)
    cfg1 = (max(1, int(0.1 * P1)), max(1, patch_sizes[1] // 4), P1)
    mask0, mask1 = pl.pallas_call(
        functools.partial(_route_kernel, cfg0=cfg0, cfg1=cfg1),
        out_shape=[jax.ShapeDtypeStruct((P0, P0), f32),
                   jax.ShapeDtypeStruct((P1, P1), f32)],
    )(params['nodevec1'], params['nodevec2'],
      r0.reshape(Bn, N, P0), r1.reshape(Bn, N, P1))

    wref0 = _stack_weights(params['encoders'][0], D, H, dh)
    wref1 = _stack_weights(params['encoders'][1], D, H, dh)
    fuse_args = [params['fusion_w'].reshape(1, -1),
                 params['fuse_W'].astype(bf16),
                 params['fuse_b'].reshape(1, D),
                 params['fin_g'].reshape(1, D),
                 params['fin_b'].reshape(1, D)]

    depth = len(params['encoders'][0])
    pooled0 = pl.pallas_call(
        functools.partial(_enc_kernel, depth=depth),
        grid=(T,),
        in_specs=[blk(G, P0, D), cst(P0, P0)] + _cst_specs(wref0),
        out_specs=blk(1, G, D),
        out_shape=jax.ShapeDtypeStruct((T, G, D), f32),
    )(patches0, mask0, *wref0)

    final = pl.pallas_call(
        functools.partial(_encfuse_kernel, depth=depth),
        grid=(T,),
        in_specs=([blk(G, P1, D), cst(P1, P1), blk(1, G, D)]
                  + _cst_specs(wref1) + _cst_specs(fuse_args)),
        out_specs=blk(1, G, D),
        out_shape=jax.ShapeDtypeStruct((T, G, D), f32),
    )(patches1, mask1, pooled0, *wref1, *fuse_args)

    return final.reshape(Bn, N, D)
